# baseline (speedup 1.0000x reference)
import functools
import jax
import jax.numpy as jnp
from jax import lax
from jax.experimental import pallas as pl
from jax.experimental.pallas import tpu as pltpu
from jax.experimental.pallas import tpu_sc as plsc

N = 10000
E = 320000
D = 128
G = 64
NP = 10240
NW = 32
CH = 96
SROWS = 3584
EP = SROWS * CH
NCH0 = 112
NCH1 = 112
C0ROWS = 16 * NCH0
EWD = SROWS // NW
RT = NP // 16
BR = 1024
GRID = NP // BR

_sc_mesh = plsc.VectorSubcoreMesh(core_axis_name="c", subcore_axis_name="s")
_sc_params = pltpu.CompilerParams(needs_layout_passes=False)


@functools.partial(
    pl.kernel,
    out_type=(
        jax.ShapeDtypeStruct((NP // 128, 128), jnp.float32),
        jax.ShapeDtypeStruct((NP // 128, 128), jnp.float32),
    ),
    mesh=_sc_mesh,
    compiler_params=_sc_params,
    scratch_types=[
        pltpu.VMEM((EWD, CH), jnp.int32),
        pltpu.VMEM((NP // 128, 128), jnp.float32),
        pltpu.VMEM((NP // 128,), jnp.int32),
        pltpu.VMEM_SHARED((NP // 128, 128), jnp.float32),
    ],
)
def _deg_kernel(dst_hbm, cntA_hbm, cntB_hbm, dstv, cnt_v, rowi_v, cnt_sh):
    c = lax.axis_index("c")
    s = lax.axis_index("s")
    wid = s * 2 + c

    def zrow(j, _):
        for k in range(8):
            cnt_v[j, pl.ds(k * 16, 16)] = jnp.zeros((16,), jnp.float32)
        return 0
    lax.fori_loop(0, NP // 128, zrow, 0)

    pltpu.sync_copy(dst_hbm.at[pl.ds(wid * EWD, EWD)], dstv)
    ones = jnp.full((16,), 1.0, jnp.float32)

    def step(j, _):
        for k in range(CH // 16):
            idx = dstv[j, pl.ds(k * 16, 16)]
            plsc.addupdate_scatter(cnt_v, [idx >> 7, idx & 127], ones)
        return 0
    lax.fori_loop(0, EWD, step, 0)

    for k in range(NP // 128 // 16):
        rowi_v[pl.ds(k * 16, 16)] = lax.iota(jnp.int32, 16) + (k * 16)

    @pl.when(s == 0)
    def _():
        pltpu.sync_copy(cnt_v, cnt_sh)
    plsc.subcore_barrier()

    @pl.when(s != 0)
    def _():
        pltpu.sync_copy(cnt_v, cnt_sh.at[rowi_v], add=True)
    plsc.subcore_barrier()

    nw_out = NP // 128 // 8
    @pl.when((c == 0) & (s < nw_out))
    def _():
        pltpu.sync_copy(cnt_sh.at[pl.ds(s * 8, 8)], cntA_hbm.at[pl.ds(s * 8, 8)])
    @pl.when((c == 1) & (s < nw_out))
    def _():
        pltpu.sync_copy(cnt_sh.at[pl.ds(s * 8, 8)], cntB_hbm.at[pl.ds(s * 8, 8)])


@functools.partial(
    pl.kernel,
    out_type=(
        jax.ShapeDtypeStruct((NP, D), jnp.float32),
        jax.ShapeDtypeStruct((NP, D), jnp.float32),
    ),
    mesh=_sc_mesh,
    compiler_params=_sc_params,
    scratch_types=[
        pltpu.VMEM((NCH0, CH), jnp.int32),
        pltpu.VMEM((CH,), jnp.int32),
        pltpu.VMEM((CH,), jnp.int32),
        pltpu.VMEM((CH, D), jnp.float32),
        pltpu.VMEM((CH, D), jnp.float32),
        pltpu.VMEM_SHARED((NP, D), jnp.float32),
        pltpu.SemaphoreType.DMA,
        pltpu.SemaphoreType.DMA,
        pltpu.SemaphoreType.DMA,
        pltpu.SemaphoreType.DMA,
    ],
)
def _edge_kernel(y_hbm, zero_hbm, src_hbm, dst_hbm, accA_hbm, accB_hbm,
                 srcv, didx0, didx1, rows0, rows1, acc_sh,
                 gsem0, gsem1, dsem0, dsem1):
    c = lax.axis_index("c")
    s = lax.axis_index("s")
    didx = (didx0, didx1)
    rows = (rows0, rows1)
    gsem = (gsem0, gsem1)
    dsem = (dsem0, dsem1)

    @pl.when(c == 0)
    def _():
        pltpu.sync_copy(y_hbm.at[pl.ds(s * RT, RT)], acc_sh.at[pl.ds(s * RT, RT)])
    @pl.when(c == 1)
    def _():
        pltpu.sync_copy(zero_hbm, acc_sh.at[pl.ds(s * RT, RT)])

    plsc.subcore_barrier()

    if NCH0 == NCH1:
        nch = NCH0
        rbase = (s * 2 + c) * NCH0
    else:
        nch = jnp.where(c == 1, NCH0, NCH1)
        rbase = jnp.where(c == 1, s * NCH0, C0ROWS + s * NCH1)

    pltpu.sync_copy(src_hbm.at[pl.ds(rbase, NCH0)], srcv)

    def gstart(q, b):
        pltpu.async_copy(y_hbm.at[srcv.at[q]], rows[b], gsem[b])

    def gwait(q, b):
        pltpu.make_async_copy(y_hbm.at[srcv.at[q]], rows[b], gsem[b]).wait()

    def dstart(q, b):
        pltpu.async_copy(dst_hbm.at[rbase + q], didx[b], dsem[b])

    def dwait(q, b):
        pltpu.make_async_copy(dst_hbm.at[rbase + q], didx[b], dsem[b]).wait()

    dstart(0, 0)
    gstart(0, 0)

    def pair(j, _):
        for b in range(2):
            q = 2 * j + b
            nb = 1 - b
            pf = jnp.minimum(q + 1, nch - 1)
            dstart(pf, nb)
            gstart(pf, nb)
            gwait(q, b)
            dwait(q, b)
            pltpu.sync_copy(rows[b], acc_sh.at[didx[b]], add=True)
        return 0
    lax.fori_loop(0, nch // 2, pair, 0)

    gwait(nch - 1, 0)
    dwait(nch - 1, 0)

    plsc.subcore_barrier()

    @pl.when(c == 0)
    def _():
        pltpu.sync_copy(acc_sh.at[pl.ds(s * RT, RT)], accA_hbm.at[pl.ds(s * RT, RT)])
    @pl.when(c == 1)
    def _():
        pltpu.sync_copy(acc_sh.at[pl.ds(s * RT, RT)], accB_hbm.at[pl.ds(s * RT, RT)])


@functools.partial(
    pl.kernel,
    out_type=(
        jax.ShapeDtypeStruct((NP // 128, 128), jnp.float32),
        jax.ShapeDtypeStruct((NP // 128, 128), jnp.float32),
    ),
    mesh=_sc_mesh,
    compiler_params=_sc_params,
    scratch_types=[
        pltpu.VMEM((EWD, CH), jnp.int32),
        pltpu.VMEM((EWD, CH), jnp.int32),
        pltpu.VMEM((NP // 128, 128), jnp.float32),
        pltpu.VMEM((NP // 128, 128), jnp.float32),
        pltpu.VMEM((NP // 128,), jnp.int32),
        pltpu.VMEM_SHARED((NP // 128, 128), jnp.float32),
    ],
)
def _scalar_kernel(y3_hbm, src_hbm, dst_hbm, sA_hbm, sB_hbm,
                   srcv, dstv, y3_v, s_v, rowi_v, s_sh):
    c = lax.axis_index("c")
    s = lax.axis_index("s")
    wid = s * 2 + c

    pltpu.sync_copy(src_hbm.at[pl.ds(wid * EWD, EWD)], srcv)
    pltpu.sync_copy(dst_hbm.at[pl.ds(wid * EWD, EWD)], dstv)
    pltpu.sync_copy(y3_hbm, y3_v)

    def zrow(j, _):
        for k in range(8):
            s_v[j, pl.ds(k * 16, 16)] = jnp.zeros((16,), jnp.float32)
        return 0
    lax.fori_loop(0, NP // 128, zrow, 0)

    def step(j, _):
        for k in range(CH // 16):
            si = srcv[j, pl.ds(k * 16, 16)]
            di = dstv[j, pl.ds(k * 16, 16)]
            vals = plsc.load_gather(y3_v, [si >> 7, si & 127])
            plsc.addupdate_scatter(s_v, [di >> 7, di & 127], vals)
        return 0
    lax.fori_loop(0, EWD, step, 0)

    for k in range(NP // 128 // 16):
        rowi_v[pl.ds(k * 16, 16)] = lax.iota(jnp.int32, 16) + (k * 16)

    @pl.when(s == 0)
    def _():
        pltpu.sync_copy(s_v, s_sh)
    plsc.subcore_barrier()

    @pl.when(s != 0)
    def _():
        pltpu.sync_copy(s_v, s_sh.at[rowi_v], add=True)
    plsc.subcore_barrier()

    nw_out = NP // 128 // 8
    @pl.when((c == 0) & (s < nw_out))
    def _():
        pltpu.sync_copy(s_sh.at[pl.ds(s * 8, 8)], sA_hbm.at[pl.ds(s * 8, 8)])
    @pl.when((c == 1) & (s < nw_out))
    def _():
        pltpu.sync_copy(s_sh.at[pl.ds(s * 8, 8)], sB_hbm.at[pl.ds(s * 8, 8)])


def _k1_body(cntA_ref, cntB_ref, x_ref, w_ref, dinv_ref, y_ref):
    cnt = cntA_ref[...] + cntB_ref[...] + 1.0
    dinv = lax.rsqrt(cnt)
    dinv_ref[...] = dinv
    y_ref[...] = dinv * jnp.dot(x_ref[...], w_ref[...],
                                preferred_element_type=jnp.float32, precision=lax.Precision.HIGHEST)


_k1 = pl.pallas_call(
    _k1_body,
    grid=(GRID,),
    in_specs=[
        pl.BlockSpec((BR, 1), lambda i: (i, 0)),
        pl.BlockSpec((BR, 1), lambda i: (i, 0)),
        pl.BlockSpec((BR, D), lambda i: (i, 0)),
        pl.BlockSpec((D, D), lambda i: (0, 0)),
    ],
    out_specs=[
        pl.BlockSpec((BR, 1), lambda i: (i, 0)),
        pl.BlockSpec((BR, D), lambda i: (i, 0)),
    ],
    out_shape=[
        jax.ShapeDtypeStruct((NP, 1), jnp.float32),
        jax.ShapeDtypeStruct((NP, D), jnp.float32),
    ],
)


def _k2_body(dinv_ref, a_ref, b_ref, bias_ref, w_ref, y_ref):
    dinv = dinv_ref[...]
    h = jnp.maximum(dinv * (a_ref[...] + b_ref[...]) + bias_ref[...], 0.0)
    y_ref[...] = dinv * jnp.dot(h, w_ref[...], preferred_element_type=jnp.float32, precision=lax.Precision.HIGHEST)


_k2 = pl.pallas_call(
    _k2_body,
    grid=(GRID,),
    in_specs=[
        pl.BlockSpec((BR, 1), lambda i: (i, 0)),
        pl.BlockSpec((BR, D), lambda i: (i, 0)),
        pl.BlockSpec((BR, D), lambda i: (i, 0)),
        pl.BlockSpec((1, D), lambda i: (0, 0)),
        pl.BlockSpec((D, D), lambda i: (0, 0)),
    ],
    out_specs=pl.BlockSpec((BR, D), lambda i: (i, 0)),
    out_shape=jax.ShapeDtypeStruct((NP, D), jnp.float32),
)


def _k3_body(dinv_ref, a_ref, b_ref, bias_ref, w3_ref, wlin_ref, y3_ref):
    dinv = dinv_ref[...]
    h = jnp.maximum(dinv * (a_ref[...] + b_ref[...]) + bias_ref[...], 0.0)
    w = jnp.dot(w3_ref[...], wlin_ref[...], preferred_element_type=jnp.float32, precision=lax.Precision.HIGHEST)
    y3_ref[...] = dinv * jnp.dot(h, w, preferred_element_type=jnp.float32, precision=lax.Precision.HIGHEST)


_k3 = pl.pallas_call(
    _k3_body,
    grid=(GRID,),
    in_specs=[
        pl.BlockSpec((BR, 1), lambda i: (i, 0)),
        pl.BlockSpec((BR, D), lambda i: (i, 0)),
        pl.BlockSpec((BR, D), lambda i: (i, 0)),
        pl.BlockSpec((1, D), lambda i: (0, 0)),
        pl.BlockSpec((D, D), lambda i: (0, 0)),
        pl.BlockSpec((D, 1), lambda i: (0, 0)),
    ],
    out_specs=pl.BlockSpec((BR, 1), lambda i: (i, 0)),
    out_shape=jax.ShapeDtypeStruct((NP, 1), jnp.float32),
)


def _pool_body(dinv_ref, sA_ref, sB_ref, y3_ref, batch_ref, b3_ref, wlt_ref,
               blin_ref, out_ref):
    i = pl.program_id(0)
    beta = jnp.sum(b3_ref[...] * wlt_ref[...])
    v = dinv_ref[...] * (sA_ref[...] + sB_ref[...] + y3_ref[...]) + beta
    gids = lax.broadcasted_iota(jnp.int32, (BR, 128), 1)
    m = batch_ref[...] == gids
    contrib = jnp.sum(jnp.where(m, v, 0.0), axis=0, keepdims=True)

    @pl.when(i == 0)
    def _():
        out_ref[...] = jnp.broadcast_to(blin_ref[...], (1, 128))
    out_ref[...] += contrib


_pool = pl.pallas_call(
    _pool_body,
    grid=(GRID,),
    in_specs=[
        pl.BlockSpec((BR, 1), lambda i: (i, 0)),
        pl.BlockSpec((BR, 1), lambda i: (i, 0)),
        pl.BlockSpec((BR, 1), lambda i: (i, 0)),
        pl.BlockSpec((BR, 1), lambda i: (i, 0)),
        pl.BlockSpec((BR, 1), lambda i: (i, 0)),
        pl.BlockSpec((1, D), lambda i: (0, 0)),
        pl.BlockSpec((1, D), lambda i: (0, 0)),
        pl.BlockSpec((1, 1), lambda i: (0, 0)),
    ],
    out_specs=pl.BlockSpec((1, 128), lambda i: (0, 0)),
    out_shape=jax.ShapeDtypeStruct((1, 128), jnp.float32),
)


def kernel(x, edge_index, batch, W1, b1, W2, b2, W3, b3, Wlin, blin):
    src = edge_index[0]
    dst = edge_index[1]
    sslack = C0ROWS + 15 * NCH1 + NCH0 - SROWS
    src2 = jnp.concatenate(
        [src, jnp.zeros((EP - E + sslack * CH,), jnp.int32)]
    ).reshape(SROWS + sslack, CH)
    dst2 = jnp.concatenate([dst, jnp.full((EP - E,), N, jnp.int32)]).reshape(SROWS, CH)
    xP = jnp.concatenate([x, jnp.zeros((NP - N, D), jnp.float32)])
    batchP = jnp.concatenate([batch, jnp.full((NP - N,), G, jnp.int32)])
    batchP = batchP.reshape(NP, 1)
    zrows = jnp.zeros((RT, D), jnp.float32)

    cntA, cntB = _deg_kernel(dst2)
    dinv, y1 = _k1(cntA.reshape(NP, 1), cntB.reshape(NP, 1), xP, W1)

    a1A, a1B = _edge_kernel(y1, zrows, src2, dst2)
    y2 = _k2(dinv, a1A, a1B, b1.reshape(1, D), W2)

    a2A, a2B = _edge_kernel(y2, zrows, src2, dst2)
    y3 = _k3(dinv, a2A, a2B, b2.reshape(1, D), W3, Wlin)

    sA, sB = _scalar_kernel(y3.reshape(NP // 128, 128), src2, dst2)
    out128 = _pool(dinv, sA.reshape(NP, 1), sB.reshape(NP, 1), y3, batchP,
                   b3.reshape(1, D), Wlin.reshape(1, D), blin.reshape(1, 1))
    return out128[0, :G].reshape(G, 1)


# speedup vs baseline: 1.0081x; 1.0081x over previous
import functools
import jax
import jax.numpy as jnp
from jax import lax
from jax.experimental import pallas as pl
from jax.experimental.pallas import tpu as pltpu
from jax.experimental.pallas import tpu_sc as plsc

N = 10000
E = 320000
D = 128
G = 64
NP = 10240
NW = 32
CH = 96
SROWS = 3584
EP = SROWS * CH
NCH0 = 112
NCH1 = 112
C0ROWS = 16 * NCH0
EWD = SROWS // NW
RT = NP // 16
BR = 1024
GRID = NP // BR

_sc_mesh = plsc.VectorSubcoreMesh(core_axis_name="c", subcore_axis_name="s")
_sc_params = pltpu.CompilerParams(needs_layout_passes=False)


@functools.partial(
    pl.kernel,
    out_type=(
        jax.ShapeDtypeStruct((NP // 128, 128), jnp.float32),
        jax.ShapeDtypeStruct((NP // 128, 128), jnp.float32),
    ),
    mesh=_sc_mesh,
    compiler_params=_sc_params,
    scratch_types=[
        pltpu.VMEM((EWD, CH), jnp.int32),
        pltpu.VMEM((NP // 128, 128), jnp.float32),
        pltpu.VMEM((NP // 128,), jnp.int32),
        pltpu.VMEM_SHARED((NP // 128, 128), jnp.float32),
    ],
)
def _deg_kernel(dst_hbm, cntA_hbm, cntB_hbm, dstv, cnt_v, rowi_v, cnt_sh):
    c = lax.axis_index("c")
    s = lax.axis_index("s")
    wid = s * 2 + c

    def zrow(j, _):
        for k in range(8):
            cnt_v[j, pl.ds(k * 16, 16)] = jnp.zeros((16,), jnp.float32)
        return 0
    lax.fori_loop(0, NP // 128, zrow, 0)

    pltpu.sync_copy(dst_hbm.at[pl.ds(wid * EWD, EWD)], dstv)
    ones = jnp.full((16,), 1.0, jnp.float32)

    def step(j, _):
        for k in range(CH // 16):
            idx = dstv[j, pl.ds(k * 16, 16)]
            plsc.addupdate_scatter(cnt_v, [idx >> 7, idx & 127], ones)
        return 0
    lax.fori_loop(0, EWD, step, 0)

    for k in range(NP // 128 // 16):
        rowi_v[pl.ds(k * 16, 16)] = lax.iota(jnp.int32, 16) + (k * 16)

    @pl.when(s == 0)
    def _():
        pltpu.sync_copy(cnt_v, cnt_sh)
    plsc.subcore_barrier()

    @pl.when(s != 0)
    def _():
        pltpu.sync_copy(cnt_v, cnt_sh.at[rowi_v], add=True)
    plsc.subcore_barrier()

    nw_out = NP // 128 // 8
    @pl.when((c == 0) & (s < nw_out))
    def _():
        pltpu.sync_copy(cnt_sh.at[pl.ds(s * 8, 8)], cntA_hbm.at[pl.ds(s * 8, 8)])
    @pl.when((c == 1) & (s < nw_out))
    def _():
        pltpu.sync_copy(cnt_sh.at[pl.ds(s * 8, 8)], cntB_hbm.at[pl.ds(s * 8, 8)])


@functools.partial(
    pl.kernel,
    out_type=(
        jax.ShapeDtypeStruct((NP, D), jnp.float32),
        jax.ShapeDtypeStruct((NP, D), jnp.float32),
    ),
    mesh=_sc_mesh,
    compiler_params=_sc_params,
    scratch_types=[
        pltpu.VMEM((NCH0, CH), jnp.int32),
        pltpu.VMEM((CH,), jnp.int32),
        pltpu.VMEM((CH,), jnp.int32),
        pltpu.VMEM((CH, D), jnp.float32),
        pltpu.VMEM((CH, D), jnp.float32),
        pltpu.VMEM_SHARED((NP, D), jnp.float32),
        pltpu.SemaphoreType.DMA,
        pltpu.SemaphoreType.DMA,
        pltpu.SemaphoreType.DMA,
        pltpu.SemaphoreType.DMA,
    ],
)
def _edge_kernel(y_hbm, zero_hbm, src_hbm, dst_hbm, accA_hbm, accB_hbm,
                 srcv, didx0, didx1, rows0, rows1, acc_sh,
                 gsem0, gsem1, dsem0, dsem1):
    c = lax.axis_index("c")
    s = lax.axis_index("s")
    didx = (didx0, didx1)
    rows = (rows0, rows1)
    gsem = (gsem0, gsem1)
    dsem = (dsem0, dsem1)

    @pl.when(c == 0)
    def _():
        pltpu.sync_copy(y_hbm.at[pl.ds(s * RT, RT)], acc_sh.at[pl.ds(s * RT, RT)])
    @pl.when(c == 1)
    def _():
        pltpu.sync_copy(zero_hbm, acc_sh.at[pl.ds(s * RT, RT)])

    plsc.subcore_barrier()

    if NCH0 == NCH1:
        nch = NCH0
        rbase = (s * 2 + c) * NCH0
    else:
        nch = jnp.where(c == 1, NCH0, NCH1)
        rbase = jnp.where(c == 1, s * NCH0, C0ROWS + s * NCH1)

    pltpu.sync_copy(src_hbm.at[pl.ds(rbase, NCH0)], srcv)

    def gstart(q, b):
        pltpu.async_copy(y_hbm.at[srcv.at[q]], rows[b], gsem[b])

    def gwait(q, b):
        pltpu.make_async_copy(y_hbm.at[srcv.at[q]], rows[b], gsem[b]).wait()

    def dstart(q, b):
        pltpu.async_copy(dst_hbm.at[rbase + q], didx[b], dsem[b])

    def dwait(q, b):
        pltpu.make_async_copy(dst_hbm.at[rbase + q], didx[b], dsem[b]).wait()

    dstart(0, 0)
    gstart(0, 0)

    def pair(j, _):
        for b in range(2):
            q = 2 * j + b
            nb = 1 - b
            pf = jnp.minimum(q + 1, nch - 1)
            dstart(pf, nb)
            gstart(pf, nb)
            gwait(q, b)
            dwait(q, b)
            pltpu.sync_copy(rows[b], acc_sh.at[didx[b]], add=True)
        return 0
    lax.fori_loop(0, nch // 2, pair, 0)

    gwait(nch - 1, 0)
    dwait(nch - 1, 0)

    plsc.subcore_barrier()

    @pl.when(c == 0)
    def _():
        pltpu.sync_copy(acc_sh.at[pl.ds(s * RT, RT)], accA_hbm.at[pl.ds(s * RT, RT)])
    @pl.when(c == 1)
    def _():
        pltpu.sync_copy(acc_sh.at[pl.ds(s * RT, RT)], accB_hbm.at[pl.ds(s * RT, RT)])


@functools.partial(
    pl.kernel,
    out_type=(
        jax.ShapeDtypeStruct((NP // 128, 128), jnp.float32),
        jax.ShapeDtypeStruct((NP // 128, 128), jnp.float32),
    ),
    mesh=_sc_mesh,
    compiler_params=_sc_params,
    scratch_types=[
        pltpu.VMEM((EWD, CH), jnp.int32),
        pltpu.VMEM((EWD, CH), jnp.int32),
        pltpu.VMEM((NP // 128, 128), jnp.float32),
        pltpu.VMEM((NP // 128, 128), jnp.float32),
        pltpu.VMEM((NP // 128,), jnp.int32),
        pltpu.VMEM_SHARED((NP // 128, 128), jnp.float32),
    ],
)
def _scalar_kernel(y3_hbm, src_hbm, dst_hbm, sA_hbm, sB_hbm,
                   srcv, dstv, y3_v, s_v, rowi_v, s_sh):
    c = lax.axis_index("c")
    s = lax.axis_index("s")
    wid = s * 2 + c

    pltpu.sync_copy(src_hbm.at[pl.ds(wid * EWD, EWD)], srcv)
    pltpu.sync_copy(dst_hbm.at[pl.ds(wid * EWD, EWD)], dstv)
    pltpu.sync_copy(y3_hbm, y3_v)

    def zrow(j, _):
        for k in range(8):
            s_v[j, pl.ds(k * 16, 16)] = jnp.zeros((16,), jnp.float32)
        return 0
    lax.fori_loop(0, NP // 128, zrow, 0)

    def step(j, _):
        for k in range(CH // 16):
            si = srcv[j, pl.ds(k * 16, 16)]
            di = dstv[j, pl.ds(k * 16, 16)]
            vals = plsc.load_gather(y3_v, [si >> 7, si & 127])
            plsc.addupdate_scatter(s_v, [di >> 7, di & 127], vals)
        return 0
    lax.fori_loop(0, EWD, step, 0)

    for k in range(NP // 128 // 16):
        rowi_v[pl.ds(k * 16, 16)] = lax.iota(jnp.int32, 16) + (k * 16)

    @pl.when(s == 0)
    def _():
        pltpu.sync_copy(s_v, s_sh)
    plsc.subcore_barrier()

    @pl.when(s != 0)
    def _():
        pltpu.sync_copy(s_v, s_sh.at[rowi_v], add=True)
    plsc.subcore_barrier()

    nw_out = NP // 128 // 8
    @pl.when((c == 0) & (s < nw_out))
    def _():
        pltpu.sync_copy(s_sh.at[pl.ds(s * 8, 8)], sA_hbm.at[pl.ds(s * 8, 8)])
    @pl.when((c == 1) & (s < nw_out))
    def _():
        pltpu.sync_copy(s_sh.at[pl.ds(s * 8, 8)], sB_hbm.at[pl.ds(s * 8, 8)])


def _k1_body(cntA_ref, cntB_ref, x_ref, w_ref, dinv_ref, y_ref):
    cnt = cntA_ref[...] + cntB_ref[...] + 1.0
    dinv = lax.rsqrt(cnt)
    dinv_ref[...] = dinv
    y_ref[...] = dinv * jnp.dot(x_ref[...], w_ref[...],
                                preferred_element_type=jnp.float32, precision=lax.Precision.HIGHEST)


_k1 = pl.pallas_call(
    _k1_body,
    grid=(GRID,),
    in_specs=[
        pl.BlockSpec((BR, 1), lambda i: (i, 0)),
        pl.BlockSpec((BR, 1), lambda i: (i, 0)),
        pl.BlockSpec((BR, D), lambda i: (i, 0)),
        pl.BlockSpec((D, D), lambda i: (0, 0)),
    ],
    out_specs=[
        pl.BlockSpec((BR, 1), lambda i: (i, 0)),
        pl.BlockSpec((BR, D), lambda i: (i, 0)),
    ],
    out_shape=[
        jax.ShapeDtypeStruct((NP, 1), jnp.float32),
        jax.ShapeDtypeStruct((NP, D), jnp.float32),
    ],
)


def _k2_body(dinv_ref, a_ref, b_ref, bias_ref, w_ref, y_ref):
    dinv = dinv_ref[...]
    h = jnp.maximum(dinv * (a_ref[...] + b_ref[...]) + bias_ref[...], 0.0)
    y_ref[...] = dinv * jnp.dot(h, w_ref[...], preferred_element_type=jnp.float32, precision=lax.Precision.HIGHEST)


_k2 = pl.pallas_call(
    _k2_body,
    grid=(GRID,),
    in_specs=[
        pl.BlockSpec((BR, 1), lambda i: (i, 0)),
        pl.BlockSpec((BR, D), lambda i: (i, 0)),
        pl.BlockSpec((BR, D), lambda i: (i, 0)),
        pl.BlockSpec((1, D), lambda i: (0, 0)),
        pl.BlockSpec((D, D), lambda i: (0, 0)),
    ],
    out_specs=pl.BlockSpec((BR, D), lambda i: (i, 0)),
    out_shape=jax.ShapeDtypeStruct((NP, D), jnp.float32),
)


def _k3_body(dinv_ref, a_ref, b_ref, bias_ref, w3_ref, wlin_ref, y3_ref):
    dinv = dinv_ref[...]
    h = jnp.maximum(dinv * (a_ref[...] + b_ref[...]) + bias_ref[...], 0.0)
    w = jnp.dot(w3_ref[...], wlin_ref[...], preferred_element_type=jnp.float32, precision=lax.Precision.HIGHEST)
    y3_ref[...] = dinv * jnp.dot(h, w, preferred_element_type=jnp.float32, precision=lax.Precision.HIGHEST)


_k3 = pl.pallas_call(
    _k3_body,
    grid=(GRID,),
    in_specs=[
        pl.BlockSpec((BR, 1), lambda i: (i, 0)),
        pl.BlockSpec((BR, D), lambda i: (i, 0)),
        pl.BlockSpec((BR, D), lambda i: (i, 0)),
        pl.BlockSpec((1, D), lambda i: (0, 0)),
        pl.BlockSpec((D, D), lambda i: (0, 0)),
        pl.BlockSpec((D, 1), lambda i: (0, 0)),
    ],
    out_specs=pl.BlockSpec((BR, 1), lambda i: (i, 0)),
    out_shape=jax.ShapeDtypeStruct((NP, 1), jnp.float32),
)


def _pool_body(dinv_ref, sA_ref, sB_ref, y3_ref, batch_ref, b3_ref, wlt_ref,
               blin_ref, out_ref):
    i = pl.program_id(0)
    beta = jnp.sum(b3_ref[...] * wlt_ref[...])
    v = dinv_ref[...] * (sA_ref[...] + sB_ref[...] + y3_ref[...]) + beta
    gids = lax.broadcasted_iota(jnp.int32, (BR, 128), 1)
    m = batch_ref[...] == gids
    contrib = jnp.sum(jnp.where(m, v, 0.0), axis=0, keepdims=True)

    @pl.when(i == 0)
    def _():
        out_ref[...] = jnp.broadcast_to(blin_ref[...], (1, 128))
    out_ref[...] += contrib


_pool = pl.pallas_call(
    _pool_body,
    grid=(GRID,),
    in_specs=[
        pl.BlockSpec((BR, 1), lambda i: (i, 0)),
        pl.BlockSpec((BR, 1), lambda i: (i, 0)),
        pl.BlockSpec((BR, 1), lambda i: (i, 0)),
        pl.BlockSpec((BR, 1), lambda i: (i, 0)),
        pl.BlockSpec((BR, 1), lambda i: (i, 0)),
        pl.BlockSpec((1, D), lambda i: (0, 0)),
        pl.BlockSpec((1, D), lambda i: (0, 0)),
        pl.BlockSpec((1, 1), lambda i: (0, 0)),
    ],
    out_specs=pl.BlockSpec((1, 128), lambda i: (0, 0)),
    out_shape=jax.ShapeDtypeStruct((1, 128), jnp.float32),
)


def kernel(x, edge_index, batch, W1, b1, W2, b2, W3, b3, Wlin, blin):
    src = edge_index[0]
    dst = edge_index[1]
    sslack = C0ROWS + 15 * NCH1 + NCH0 - SROWS
    src2 = jnp.concatenate(
        [src, jnp.zeros((EP - E + sslack * CH,), jnp.int32)]
    ).reshape(SROWS + sslack, CH)
    pad_dst = N + jnp.arange(EP - E, dtype=jnp.int32) % (NP - N)
    dst2 = jnp.concatenate([dst, pad_dst]).reshape(SROWS, CH)
    xP = jnp.concatenate([x, jnp.zeros((NP - N, D), jnp.float32)])
    batchP = jnp.concatenate([batch, jnp.full((NP - N,), G, jnp.int32)])
    batchP = batchP.reshape(NP, 1)
    zrows = jnp.zeros((RT, D), jnp.float32)

    cntA, cntB = _deg_kernel(dst2)
    dinv, y1 = _k1(cntA.reshape(NP, 1), cntB.reshape(NP, 1), xP, W1)

    a1A, a1B = _edge_kernel(y1, zrows, src2, dst2)
    y2 = _k2(dinv, a1A, a1B, b1.reshape(1, D), W2)

    a2A, a2B = _edge_kernel(y2, zrows, src2, dst2)
    y3 = _k3(dinv, a2A, a2B, b2.reshape(1, D), W3, Wlin)

    sA, sB = _scalar_kernel(y3.reshape(NP // 128, 128), src2, dst2)
    out128 = _pool(dinv, sA.reshape(NP, 1), sB.reshape(NP, 1), y3, batchP,
                   b3.reshape(1, D), Wlin.reshape(1, D), blin.reshape(1, 1))
    return out128[0, :G].reshape(G, 1)


# speedup vs baseline: 1.8055x; 1.7910x over previous
import functools
import jax
import jax.numpy as jnp
from jax import lax
from jax.experimental import pallas as pl
from jax.experimental.pallas import tpu as pltpu
from jax.experimental.pallas import tpu_sc as plsc

N = 10000
E = 320000
D = 128
G = 64
NP = 10240
NW = 32
CH = 128
SROWS = 2560
EP = SROWS * CH
NCH0 = 80
NCH1 = 80
C0ROWS = 16 * NCH0
EWD = SROWS // NW
RT = NP // 16
BR = 1024
GRID = NP // BR

_sc_mesh = plsc.VectorSubcoreMesh(core_axis_name="c", subcore_axis_name="s")
_sc_params = pltpu.CompilerParams(needs_layout_passes=False)


@functools.partial(
    pl.kernel,
    out_type=(
        jax.ShapeDtypeStruct((NP // 128, 128), jnp.float32),
        jax.ShapeDtypeStruct((NP // 128, 128), jnp.float32),
    ),
    mesh=_sc_mesh,
    compiler_params=_sc_params,
    scratch_types=[
        pltpu.VMEM((EWD, CH), jnp.int32),
        pltpu.VMEM((NP // 128, 128), jnp.float32),
        pltpu.VMEM((NP // 128,), jnp.int32),
        pltpu.VMEM_SHARED((NP // 128, 128), jnp.float32),
    ],
)
def _deg_kernel(dst_hbm, cntA_hbm, cntB_hbm, dstv, cnt_v, rowi_v, cnt_sh):
    c = lax.axis_index("c")
    s = lax.axis_index("s")
    wid = s * 2 + c

    def zrow(j, _):
        for k in range(8):
            cnt_v[j, pl.ds(k * 16, 16)] = jnp.zeros((16,), jnp.float32)
        return 0
    lax.fori_loop(0, NP // 128, zrow, 0)

    pltpu.sync_copy(dst_hbm.at[pl.ds(wid * EWD, EWD)], dstv)
    ones = jnp.full((16,), 1.0, jnp.float32)

    def step(j, _):
        for k in range(CH // 16):
            idx = dstv[j, pl.ds(k * 16, 16)]
            plsc.addupdate_scatter(cnt_v, [idx >> 7, idx & 127], ones)
        return 0
    lax.fori_loop(0, EWD, step, 0)

    for k in range(NP // 128 // 16):
        rowi_v[pl.ds(k * 16, 16)] = lax.iota(jnp.int32, 16) + (k * 16)

    @pl.when(s == 0)
    def _():
        pltpu.sync_copy(cnt_v, cnt_sh)
    plsc.subcore_barrier()

    @pl.when(s != 0)
    def _():
        pltpu.sync_copy(cnt_v, cnt_sh.at[rowi_v], add=True)
    plsc.subcore_barrier()

    nw_out = NP // 128 // 8
    @pl.when((c == 0) & (s < nw_out))
    def _():
        pltpu.sync_copy(cnt_sh.at[pl.ds(s * 8, 8)], cntA_hbm.at[pl.ds(s * 8, 8)])
    @pl.when((c == 1) & (s < nw_out))
    def _():
        pltpu.sync_copy(cnt_sh.at[pl.ds(s * 8, 8)], cntB_hbm.at[pl.ds(s * 8, 8)])


@functools.partial(
    pl.kernel,
    out_type=(
        jax.ShapeDtypeStruct((NP, D), jnp.float32),
        jax.ShapeDtypeStruct((NP, D), jnp.float32),
    ),
    mesh=_sc_mesh,
    compiler_params=_sc_params,
    scratch_types=[
        pltpu.VMEM((NCH0, CH), jnp.int32),
        pltpu.VMEM((CH,), jnp.int32),
        pltpu.VMEM((CH,), jnp.int32),
        pltpu.VMEM((CH, D), jnp.float32),
        pltpu.VMEM((CH, D), jnp.float32),
        pltpu.VMEM_SHARED((NP, D), jnp.float32),
        pltpu.SemaphoreType.DMA,
        pltpu.SemaphoreType.DMA,
        pltpu.SemaphoreType.DMA,
        pltpu.SemaphoreType.DMA,
    ],
)
def _edge_kernel(y_hbm, zero_hbm, src_hbm, dst_hbm, accA_hbm, accB_hbm,
                 srcv, didx0, didx1, rows0, rows1, acc_sh,
                 gsem0, gsem1, dsem0, dsem1):
    c = lax.axis_index("c")
    s = lax.axis_index("s")
    didx = (didx0, didx1)
    rows = (rows0, rows1)
    gsem = (gsem0, gsem1)
    dsem = (dsem0, dsem1)

    @pl.when(c == 0)
    def _():
        pltpu.sync_copy(y_hbm.at[pl.ds(s * RT, RT)], acc_sh.at[pl.ds(s * RT, RT)])
    @pl.when(c == 1)
    def _():
        pltpu.sync_copy(zero_hbm, acc_sh.at[pl.ds(s * RT, RT)])

    plsc.subcore_barrier()

    if NCH0 == NCH1:
        nch = NCH0
        rbase = (s * 2 + c) * NCH0
    else:
        nch = jnp.where(c == 1, NCH0, NCH1)
        rbase = jnp.where(c == 1, s * NCH0, C0ROWS + s * NCH1)

    pltpu.sync_copy(src_hbm.at[pl.ds(rbase, NCH0)], srcv)

    def gstart(q, b):
        pltpu.async_copy(y_hbm.at[srcv.at[q]], rows[b], gsem[b])

    def gwait(q, b):
        pltpu.make_async_copy(y_hbm.at[srcv.at[q]], rows[b], gsem[b]).wait()

    def dstart(q, b):
        pltpu.async_copy(dst_hbm.at[rbase + q], didx[b], dsem[b])

    def dwait(q, b):
        pltpu.make_async_copy(dst_hbm.at[rbase + q], didx[b], dsem[b]).wait()

    dstart(0, 0)
    gstart(0, 0)

    def pair(j, _):
        for b in range(2):
            q = 2 * j + b
            nb = 1 - b

            @pl.when(q + 1 < nch)
            def _():
                dstart(q + 1, nb)
                gstart(q + 1, nb)

            gwait(q, b)
            dwait(q, b)
            pltpu.sync_copy(rows[b], acc_sh.at[didx[b]], add=True)
        return 0
    lax.fori_loop(0, nch // 2, pair, 0)

    plsc.subcore_barrier()

    @pl.when(c == 0)
    def _():
        pltpu.sync_copy(acc_sh.at[pl.ds(s * RT, RT)], accA_hbm.at[pl.ds(s * RT, RT)])
    @pl.when(c == 1)
    def _():
        pltpu.sync_copy(acc_sh.at[pl.ds(s * RT, RT)], accB_hbm.at[pl.ds(s * RT, RT)])


@functools.partial(
    pl.kernel,
    out_type=(
        jax.ShapeDtypeStruct((NP // 128, 128), jnp.float32),
        jax.ShapeDtypeStruct((NP // 128, 128), jnp.float32),
    ),
    mesh=_sc_mesh,
    compiler_params=_sc_params,
    scratch_types=[
        pltpu.VMEM((EWD, CH), jnp.int32),
        pltpu.VMEM((EWD, CH), jnp.int32),
        pltpu.VMEM((NP // 128, 128), jnp.float32),
        pltpu.VMEM((NP // 128, 128), jnp.float32),
        pltpu.VMEM((NP // 128,), jnp.int32),
        pltpu.VMEM_SHARED((NP // 128, 128), jnp.float32),
    ],
)
def _scalar_kernel(y3_hbm, src_hbm, dst_hbm, sA_hbm, sB_hbm,
                   srcv, dstv, y3_v, s_v, rowi_v, s_sh):
    c = lax.axis_index("c")
    s = lax.axis_index("s")
    wid = s * 2 + c

    pltpu.sync_copy(src_hbm.at[pl.ds(wid * EWD, EWD)], srcv)
    pltpu.sync_copy(dst_hbm.at[pl.ds(wid * EWD, EWD)], dstv)
    pltpu.sync_copy(y3_hbm, y3_v)

    def zrow(j, _):
        for k in range(8):
            s_v[j, pl.ds(k * 16, 16)] = jnp.zeros((16,), jnp.float32)
        return 0
    lax.fori_loop(0, NP // 128, zrow, 0)

    def step(j, _):
        for k in range(CH // 16):
            si = srcv[j, pl.ds(k * 16, 16)]
            di = dstv[j, pl.ds(k * 16, 16)]
            vals = plsc.load_gather(y3_v, [si >> 7, si & 127])
            plsc.addupdate_scatter(s_v, [di >> 7, di & 127], vals)
        return 0
    lax.fori_loop(0, EWD, step, 0)

    for k in range(NP // 128 // 16):
        rowi_v[pl.ds(k * 16, 16)] = lax.iota(jnp.int32, 16) + (k * 16)

    @pl.when(s == 0)
    def _():
        pltpu.sync_copy(s_v, s_sh)
    plsc.subcore_barrier()

    @pl.when(s != 0)
    def _():
        pltpu.sync_copy(s_v, s_sh.at[rowi_v], add=True)
    plsc.subcore_barrier()

    nw_out = NP // 128 // 8
    @pl.when((c == 0) & (s < nw_out))
    def _():
        pltpu.sync_copy(s_sh.at[pl.ds(s * 8, 8)], sA_hbm.at[pl.ds(s * 8, 8)])
    @pl.when((c == 1) & (s < nw_out))
    def _():
        pltpu.sync_copy(s_sh.at[pl.ds(s * 8, 8)], sB_hbm.at[pl.ds(s * 8, 8)])


def _k1_body(cntA_ref, cntB_ref, x_ref, w_ref, dinv_ref, y_ref):
    cnt = cntA_ref[...] + cntB_ref[...] + 1.0
    dinv = lax.rsqrt(cnt)
    dinv_ref[...] = dinv
    y_ref[...] = dinv * jnp.dot(x_ref[...], w_ref[...],
                                preferred_element_type=jnp.float32, precision=lax.Precision.HIGHEST)


_k1 = pl.pallas_call(
    _k1_body,
    grid=(GRID,),
    in_specs=[
        pl.BlockSpec((BR, 1), lambda i: (i, 0)),
        pl.BlockSpec((BR, 1), lambda i: (i, 0)),
        pl.BlockSpec((BR, D), lambda i: (i, 0)),
        pl.BlockSpec((D, D), lambda i: (0, 0)),
    ],
    out_specs=[
        pl.BlockSpec((BR, 1), lambda i: (i, 0)),
        pl.BlockSpec((BR, D), lambda i: (i, 0)),
    ],
    out_shape=[
        jax.ShapeDtypeStruct((NP, 1), jnp.float32),
        jax.ShapeDtypeStruct((NP, D), jnp.float32),
    ],
)


def _k2_body(dinv_ref, a_ref, b_ref, bias_ref, w_ref, y_ref):
    dinv = dinv_ref[...]
    h = jnp.maximum(dinv * (a_ref[...] + b_ref[...]) + bias_ref[...], 0.0)
    y_ref[...] = dinv * jnp.dot(h, w_ref[...], preferred_element_type=jnp.float32, precision=lax.Precision.HIGHEST)


_k2 = pl.pallas_call(
    _k2_body,
    grid=(GRID,),
    in_specs=[
        pl.BlockSpec((BR, 1), lambda i: (i, 0)),
        pl.BlockSpec((BR, D), lambda i: (i, 0)),
        pl.BlockSpec((BR, D), lambda i: (i, 0)),
        pl.BlockSpec((1, D), lambda i: (0, 0)),
        pl.BlockSpec((D, D), lambda i: (0, 0)),
    ],
    out_specs=pl.BlockSpec((BR, D), lambda i: (i, 0)),
    out_shape=jax.ShapeDtypeStruct((NP, D), jnp.float32),
)


def _k3_body(dinv_ref, a_ref, b_ref, bias_ref, w3_ref, wlin_ref, y3_ref):
    dinv = dinv_ref[...]
    h = jnp.maximum(dinv * (a_ref[...] + b_ref[...]) + bias_ref[...], 0.0)
    w = jnp.dot(w3_ref[...], wlin_ref[...], preferred_element_type=jnp.float32, precision=lax.Precision.HIGHEST)
    y3_ref[...] = dinv * jnp.dot(h, w, preferred_element_type=jnp.float32, precision=lax.Precision.HIGHEST)


_k3 = pl.pallas_call(
    _k3_body,
    grid=(GRID,),
    in_specs=[
        pl.BlockSpec((BR, 1), lambda i: (i, 0)),
        pl.BlockSpec((BR, D), lambda i: (i, 0)),
        pl.BlockSpec((BR, D), lambda i: (i, 0)),
        pl.BlockSpec((1, D), lambda i: (0, 0)),
        pl.BlockSpec((D, D), lambda i: (0, 0)),
        pl.BlockSpec((D, 1), lambda i: (0, 0)),
    ],
    out_specs=pl.BlockSpec((BR, 1), lambda i: (i, 0)),
    out_shape=jax.ShapeDtypeStruct((NP, 1), jnp.float32),
)


def _pool_body(dinv_ref, sA_ref, sB_ref, y3_ref, batch_ref, b3_ref, wlt_ref,
               blin_ref, out_ref):
    i = pl.program_id(0)
    beta = jnp.sum(b3_ref[...] * wlt_ref[...])
    v = dinv_ref[...] * (sA_ref[...] + sB_ref[...] + y3_ref[...]) + beta
    gids = lax.broadcasted_iota(jnp.int32, (BR, 128), 1)
    m = batch_ref[...] == gids
    contrib = jnp.sum(jnp.where(m, v, 0.0), axis=0, keepdims=True)

    @pl.when(i == 0)
    def _():
        out_ref[...] = jnp.broadcast_to(blin_ref[...], (1, 128))
    out_ref[...] += contrib


_pool = pl.pallas_call(
    _pool_body,
    grid=(GRID,),
    in_specs=[
        pl.BlockSpec((BR, 1), lambda i: (i, 0)),
        pl.BlockSpec((BR, 1), lambda i: (i, 0)),
        pl.BlockSpec((BR, 1), lambda i: (i, 0)),
        pl.BlockSpec((BR, 1), lambda i: (i, 0)),
        pl.BlockSpec((BR, 1), lambda i: (i, 0)),
        pl.BlockSpec((1, D), lambda i: (0, 0)),
        pl.BlockSpec((1, D), lambda i: (0, 0)),
        pl.BlockSpec((1, 1), lambda i: (0, 0)),
    ],
    out_specs=pl.BlockSpec((1, 128), lambda i: (0, 0)),
    out_shape=jax.ShapeDtypeStruct((1, 128), jnp.float32),
)


def kernel(x, edge_index, batch, W1, b1, W2, b2, W3, b3, Wlin, blin):
    src = edge_index[0]
    dst = edge_index[1]
    sslack = C0ROWS + 15 * NCH1 + NCH0 - SROWS
    src2 = jnp.concatenate(
        [src, jnp.zeros((EP - E + sslack * CH,), jnp.int32)]
    ).reshape(SROWS + sslack, CH)
    pad_dst = N + jnp.arange(EP - E, dtype=jnp.int32) % (NP - N)
    dst2 = jnp.concatenate([dst, pad_dst]).reshape(SROWS, CH)
    xP = jnp.concatenate([x, jnp.zeros((NP - N, D), jnp.float32)])
    batchP = jnp.concatenate([batch, jnp.full((NP - N,), G, jnp.int32)])
    batchP = batchP.reshape(NP, 1)
    zrows = jnp.zeros((RT, D), jnp.float32)

    cntA, cntB = _deg_kernel(dst2)
    dinv, y1 = _k1(cntA.reshape(NP, 1), cntB.reshape(NP, 1), xP, W1)

    a1A, a1B = _edge_kernel(y1, zrows, src2, dst2)
    y2 = _k2(dinv, a1A, a1B, b1.reshape(1, D), W2)

    a2A, a2B = _edge_kernel(y2, zrows, src2, dst2)
    y3 = _k3(dinv, a2A, a2B, b2.reshape(1, D), W3, Wlin)

    sA, sB = _scalar_kernel(y3.reshape(NP // 128, 128), src2, dst2)
    out128 = _pool(dinv, sA.reshape(NP, 1), sB.reshape(NP, 1), y3, batchP,
                   b3.reshape(1, D), Wlin.reshape(1, D), blin.reshape(1, 1))
    return out128[0, :G].reshape(G, 1)


# speedup vs baseline: 2.6130x; 1.4473x over previous
import functools
import jax
import jax.numpy as jnp
from jax import lax
from jax.experimental import pallas as pl
from jax.experimental.pallas import tpu as pltpu
from jax.experimental.pallas import tpu_sc as plsc

N = 10000
E = 320000
D = 128
G = 64
NP = 10240
NW = 32
CH = 128
SROWS = 2560
EP = SROWS * CH
NCH0 = 120
NCH1 = 40
C0ROWS = 16 * NCH0
EWD = SROWS // NW
RT = NP // 16
BR = 1024
GRID = NP // BR

_sc_mesh = plsc.VectorSubcoreMesh(core_axis_name="c", subcore_axis_name="s")
_sc_params = pltpu.CompilerParams(needs_layout_passes=False)


@functools.partial(
    pl.kernel,
    out_type=(
        jax.ShapeDtypeStruct((NP // 128, 128), jnp.float32),
        jax.ShapeDtypeStruct((NP // 128, 128), jnp.float32),
    ),
    mesh=_sc_mesh,
    compiler_params=_sc_params,
    scratch_types=[
        pltpu.VMEM((EWD, CH), jnp.int32),
        pltpu.VMEM((NP // 128, 128), jnp.float32),
        pltpu.VMEM((NP // 128,), jnp.int32),
        pltpu.VMEM_SHARED((NP // 128, 128), jnp.float32),
    ],
)
def _deg_kernel(dst_hbm, cntA_hbm, cntB_hbm, dstv, cnt_v, rowi_v, cnt_sh):
    c = lax.axis_index("c")
    s = lax.axis_index("s")
    wid = s * 2 + c

    def zrow(j, _):
        for k in range(8):
            cnt_v[j, pl.ds(k * 16, 16)] = jnp.zeros((16,), jnp.float32)
        return 0
    lax.fori_loop(0, NP // 128, zrow, 0)

    pltpu.sync_copy(dst_hbm.at[pl.ds(wid * EWD, EWD)], dstv)
    ones = jnp.full((16,), 1.0, jnp.float32)

    def step(j, _):
        for k in range(CH // 16):
            idx = dstv[j, pl.ds(k * 16, 16)]
            plsc.addupdate_scatter(cnt_v, [idx >> 7, idx & 127], ones)
        return 0
    lax.fori_loop(0, EWD, step, 0)

    for k in range(NP // 128 // 16):
        rowi_v[pl.ds(k * 16, 16)] = lax.iota(jnp.int32, 16) + (k * 16)

    @pl.when(s == 0)
    def _():
        pltpu.sync_copy(cnt_v, cnt_sh)
    plsc.subcore_barrier()

    @pl.when(s != 0)
    def _():
        pltpu.sync_copy(cnt_v, cnt_sh.at[rowi_v], add=True)
    plsc.subcore_barrier()

    nw_out = NP // 128 // 8
    @pl.when((c == 0) & (s < nw_out))
    def _():
        pltpu.sync_copy(cnt_sh.at[pl.ds(s * 8, 8)], cntA_hbm.at[pl.ds(s * 8, 8)])
    @pl.when((c == 1) & (s < nw_out))
    def _():
        pltpu.sync_copy(cnt_sh.at[pl.ds(s * 8, 8)], cntB_hbm.at[pl.ds(s * 8, 8)])


@functools.partial(
    pl.kernel,
    out_type=(
        jax.ShapeDtypeStruct((NP, D), jnp.float32),
        jax.ShapeDtypeStruct((NP, D), jnp.float32),
    ),
    mesh=_sc_mesh,
    compiler_params=_sc_params,
    scratch_types=[
        pltpu.VMEM((NCH0, CH), jnp.int32),
        pltpu.VMEM((CH,), jnp.int32),
        pltpu.VMEM((CH,), jnp.int32),
        pltpu.VMEM((CH, D), jnp.float32),
        pltpu.VMEM((CH, D), jnp.float32),
        pltpu.VMEM_SHARED((NP, D), jnp.float32),
        pltpu.SemaphoreType.DMA,
        pltpu.SemaphoreType.DMA,
        pltpu.SemaphoreType.DMA,
        pltpu.SemaphoreType.DMA,
    ],
)
def _edge_kernel(y_hbm, zero_hbm, src_hbm, dst_hbm, accA_hbm, accB_hbm,
                 srcv, didx0, didx1, rows0, rows1, acc_sh,
                 gsem0, gsem1, dsem0, dsem1):
    c = lax.axis_index("c")
    s = lax.axis_index("s")
    didx = (didx0, didx1)
    rows = (rows0, rows1)
    gsem = (gsem0, gsem1)
    dsem = (dsem0, dsem1)

    @pl.when(c == 0)
    def _():
        pltpu.sync_copy(y_hbm.at[pl.ds(s * RT, RT)], acc_sh.at[pl.ds(s * RT, RT)])
    @pl.when(c == 1)
    def _():
        pltpu.sync_copy(zero_hbm, acc_sh.at[pl.ds(s * RT, RT)])

    plsc.subcore_barrier()

    if NCH0 == NCH1:
        nch = NCH0
        rbase = (s * 2 + c) * NCH0
    else:
        nch = jnp.where(c == 1, NCH0, NCH1)
        rbase = jnp.where(c == 1, s * NCH0, C0ROWS + s * NCH1)

    pltpu.sync_copy(src_hbm.at[pl.ds(rbase, NCH0)], srcv)

    def gstart(q, b):
        pltpu.async_copy(y_hbm.at[srcv.at[q]], rows[b], gsem[b])

    def gwait(q, b):
        pltpu.make_async_copy(y_hbm.at[srcv.at[q]], rows[b], gsem[b]).wait()

    def dstart(q, b):
        pltpu.async_copy(dst_hbm.at[rbase + q], didx[b], dsem[b])

    def dwait(q, b):
        pltpu.make_async_copy(dst_hbm.at[rbase + q], didx[b], dsem[b]).wait()

    dstart(0, 0)
    gstart(0, 0)

    def pair(j, _):
        for b in range(2):
            q = 2 * j + b
            nb = 1 - b

            @pl.when(q + 1 < nch)
            def _():
                dstart(q + 1, nb)
                gstart(q + 1, nb)

            gwait(q, b)
            dwait(q, b)
            pltpu.sync_copy(rows[b], acc_sh.at[didx[b]], add=True)
        return 0
    lax.fori_loop(0, nch // 2, pair, 0)

    plsc.subcore_barrier()

    @pl.when(c == 0)
    def _():
        pltpu.sync_copy(acc_sh.at[pl.ds(s * RT, RT)], accA_hbm.at[pl.ds(s * RT, RT)])
    @pl.when(c == 1)
    def _():
        pltpu.sync_copy(acc_sh.at[pl.ds(s * RT, RT)], accB_hbm.at[pl.ds(s * RT, RT)])


@functools.partial(
    pl.kernel,
    out_type=(
        jax.ShapeDtypeStruct((NP // 128, 128), jnp.float32),
        jax.ShapeDtypeStruct((NP // 128, 128), jnp.float32),
    ),
    mesh=_sc_mesh,
    compiler_params=_sc_params,
    scratch_types=[
        pltpu.VMEM((EWD, CH), jnp.int32),
        pltpu.VMEM((EWD, CH), jnp.int32),
        pltpu.VMEM((NP // 128, 128), jnp.float32),
        pltpu.VMEM((NP // 128, 128), jnp.float32),
        pltpu.VMEM((NP // 128,), jnp.int32),
        pltpu.VMEM_SHARED((NP // 128, 128), jnp.float32),
    ],
)
def _scalar_kernel(y3_hbm, src_hbm, dst_hbm, sA_hbm, sB_hbm,
                   srcv, dstv, y3_v, s_v, rowi_v, s_sh):
    c = lax.axis_index("c")
    s = lax.axis_index("s")
    wid = s * 2 + c

    pltpu.sync_copy(src_hbm.at[pl.ds(wid * EWD, EWD)], srcv)
    pltpu.sync_copy(dst_hbm.at[pl.ds(wid * EWD, EWD)], dstv)
    pltpu.sync_copy(y3_hbm, y3_v)

    def zrow(j, _):
        for k in range(8):
            s_v[j, pl.ds(k * 16, 16)] = jnp.zeros((16,), jnp.float32)
        return 0
    lax.fori_loop(0, NP // 128, zrow, 0)

    def step(j, _):
        for k in range(CH // 16):
            si = srcv[j, pl.ds(k * 16, 16)]
            di = dstv[j, pl.ds(k * 16, 16)]
            vals = plsc.load_gather(y3_v, [si >> 7, si & 127])
            plsc.addupdate_scatter(s_v, [di >> 7, di & 127], vals)
        return 0
    lax.fori_loop(0, EWD, step, 0)

    for k in range(NP // 128 // 16):
        rowi_v[pl.ds(k * 16, 16)] = lax.iota(jnp.int32, 16) + (k * 16)

    @pl.when(s == 0)
    def _():
        pltpu.sync_copy(s_v, s_sh)
    plsc.subcore_barrier()

    @pl.when(s != 0)
    def _():
        pltpu.sync_copy(s_v, s_sh.at[rowi_v], add=True)
    plsc.subcore_barrier()

    nw_out = NP // 128 // 8
    @pl.when((c == 0) & (s < nw_out))
    def _():
        pltpu.sync_copy(s_sh.at[pl.ds(s * 8, 8)], sA_hbm.at[pl.ds(s * 8, 8)])
    @pl.when((c == 1) & (s < nw_out))
    def _():
        pltpu.sync_copy(s_sh.at[pl.ds(s * 8, 8)], sB_hbm.at[pl.ds(s * 8, 8)])


def _k1_body(cntA_ref, cntB_ref, x_ref, w_ref, dinv_ref, y_ref):
    cnt = cntA_ref[...] + cntB_ref[...] + 1.0
    dinv = lax.rsqrt(cnt)
    dinv_ref[...] = dinv
    y_ref[...] = dinv * jnp.dot(x_ref[...], w_ref[...],
                                preferred_element_type=jnp.float32, precision=lax.Precision.HIGHEST)


_k1 = pl.pallas_call(
    _k1_body,
    grid=(GRID,),
    in_specs=[
        pl.BlockSpec((BR, 1), lambda i: (i, 0)),
        pl.BlockSpec((BR, 1), lambda i: (i, 0)),
        pl.BlockSpec((BR, D), lambda i: (i, 0)),
        pl.BlockSpec((D, D), lambda i: (0, 0)),
    ],
    out_specs=[
        pl.BlockSpec((BR, 1), lambda i: (i, 0)),
        pl.BlockSpec((BR, D), lambda i: (i, 0)),
    ],
    out_shape=[
        jax.ShapeDtypeStruct((NP, 1), jnp.float32),
        jax.ShapeDtypeStruct((NP, D), jnp.float32),
    ],
)


def _k2_body(dinv_ref, a_ref, b_ref, bias_ref, w_ref, y_ref):
    dinv = dinv_ref[...]
    h = jnp.maximum(dinv * (a_ref[...] + b_ref[...]) + bias_ref[...], 0.0)
    y_ref[...] = dinv * jnp.dot(h, w_ref[...], preferred_element_type=jnp.float32, precision=lax.Precision.HIGHEST)


_k2 = pl.pallas_call(
    _k2_body,
    grid=(GRID,),
    in_specs=[
        pl.BlockSpec((BR, 1), lambda i: (i, 0)),
        pl.BlockSpec((BR, D), lambda i: (i, 0)),
        pl.BlockSpec((BR, D), lambda i: (i, 0)),
        pl.BlockSpec((1, D), lambda i: (0, 0)),
        pl.BlockSpec((D, D), lambda i: (0, 0)),
    ],
    out_specs=pl.BlockSpec((BR, D), lambda i: (i, 0)),
    out_shape=jax.ShapeDtypeStruct((NP, D), jnp.float32),
)


def _k3_body(dinv_ref, a_ref, b_ref, bias_ref, w3_ref, wlin_ref, y3_ref):
    dinv = dinv_ref[...]
    h = jnp.maximum(dinv * (a_ref[...] + b_ref[...]) + bias_ref[...], 0.0)
    w = jnp.dot(w3_ref[...], wlin_ref[...], preferred_element_type=jnp.float32, precision=lax.Precision.HIGHEST)
    y3_ref[...] = dinv * jnp.dot(h, w, preferred_element_type=jnp.float32, precision=lax.Precision.HIGHEST)


_k3 = pl.pallas_call(
    _k3_body,
    grid=(GRID,),
    in_specs=[
        pl.BlockSpec((BR, 1), lambda i: (i, 0)),
        pl.BlockSpec((BR, D), lambda i: (i, 0)),
        pl.BlockSpec((BR, D), lambda i: (i, 0)),
        pl.BlockSpec((1, D), lambda i: (0, 0)),
        pl.BlockSpec((D, D), lambda i: (0, 0)),
        pl.BlockSpec((D, 1), lambda i: (0, 0)),
    ],
    out_specs=pl.BlockSpec((BR, 1), lambda i: (i, 0)),
    out_shape=jax.ShapeDtypeStruct((NP, 1), jnp.float32),
)


def _pool_body(dinv_ref, sA_ref, sB_ref, y3_ref, batch_ref, b3_ref, wlt_ref,
               blin_ref, out_ref):
    i = pl.program_id(0)
    beta = jnp.sum(b3_ref[...] * wlt_ref[...])
    v = dinv_ref[...] * (sA_ref[...] + sB_ref[...] + y3_ref[...]) + beta
    gids = lax.broadcasted_iota(jnp.int32, (BR, 128), 1)
    m = batch_ref[...] == gids
    contrib = jnp.sum(jnp.where(m, v, 0.0), axis=0, keepdims=True)

    @pl.when(i == 0)
    def _():
        out_ref[...] = jnp.broadcast_to(blin_ref[...], (1, 128))
    out_ref[...] += contrib


_pool = pl.pallas_call(
    _pool_body,
    grid=(GRID,),
    in_specs=[
        pl.BlockSpec((BR, 1), lambda i: (i, 0)),
        pl.BlockSpec((BR, 1), lambda i: (i, 0)),
        pl.BlockSpec((BR, 1), lambda i: (i, 0)),
        pl.BlockSpec((BR, 1), lambda i: (i, 0)),
        pl.BlockSpec((BR, 1), lambda i: (i, 0)),
        pl.BlockSpec((1, D), lambda i: (0, 0)),
        pl.BlockSpec((1, D), lambda i: (0, 0)),
        pl.BlockSpec((1, 1), lambda i: (0, 0)),
    ],
    out_specs=pl.BlockSpec((1, 128), lambda i: (0, 0)),
    out_shape=jax.ShapeDtypeStruct((1, 128), jnp.float32),
)


def kernel(x, edge_index, batch, W1, b1, W2, b2, W3, b3, Wlin, blin):
    src = edge_index[0]
    dst = edge_index[1]
    sslack = C0ROWS + 15 * NCH1 + NCH0 - SROWS
    src2 = jnp.concatenate(
        [src, jnp.zeros((EP - E + sslack * CH,), jnp.int32)]
    ).reshape(SROWS + sslack, CH)
    pad_dst = N + jnp.arange(EP - E, dtype=jnp.int32) % (NP - N)
    dst2 = jnp.concatenate([dst, pad_dst]).reshape(SROWS, CH)
    xP = jnp.concatenate([x, jnp.zeros((NP - N, D), jnp.float32)])
    batchP = jnp.concatenate([batch, jnp.full((NP - N,), G, jnp.int32)])
    batchP = batchP.reshape(NP, 1)
    zrows = jnp.zeros((RT, D), jnp.float32)

    cntA, cntB = _deg_kernel(dst2)
    dinv, y1 = _k1(cntA.reshape(NP, 1), cntB.reshape(NP, 1), xP, W1)

    a1A, a1B = _edge_kernel(y1, zrows, src2, dst2)
    y2 = _k2(dinv, a1A, a1B, b1.reshape(1, D), W2)

    a2A, a2B = _edge_kernel(y2, zrows, src2, dst2)
    y3 = _k3(dinv, a2A, a2B, b2.reshape(1, D), W3, Wlin)

    sA, sB = _scalar_kernel(y3.reshape(NP // 128, 128), src2, dst2)
    out128 = _pool(dinv, sA.reshape(NP, 1), sB.reshape(NP, 1), y3, batchP,
                   b3.reshape(1, D), Wlin.reshape(1, D), blin.reshape(1, 1))
    return out128[0, :G].reshape(G, 1)
